# baseline (speedup 1.0000x reference)
import functools

import jax
import jax.numpy as jnp
from jax import lax
from jax.experimental import pallas as pl
from jax.experimental.pallas import tpu as pltpu
from jax.experimental.pallas import tpu_sc as plsc

N = 320000
D = 128
L = 16
NC = 2
NS = 16
NW = NC * NS

SC_ROWS = 128000
ROWS_PER_W = SC_ROWS // NW
CHUNK = 200
NCHUNK = ROWS_PER_W // CHUNK

TC_ROWS = N - SC_ROWS
TC_BLK = 3200
TC_GRID = TC_ROWS // TC_BLK
TC_START_BLK = SC_ROWS // TC_BLK

_mesh = plsc.VectorSubcoreMesh(core_axis_name="c", subcore_axis_name="s")


@functools.partial(
    pl.kernel,
    mesh=_mesh,
    out_type=jax.ShapeDtypeStruct((NW * D,), jnp.float32),
    scratch_types=[
        pltpu.VMEM((2, CHUNK, D), jnp.float32),
        pltpu.VMEM((D,), jnp.float32),
        pltpu.SemaphoreType.DMA,
        pltpu.SemaphoreType.DMA,
    ],
)
def _sc_partial_sums(data_hbm, out_hbm, buf, accv, sem0, sem1):
    wid = lax.axis_index("s") * NC + lax.axis_index("c")
    base = wid * ROWS_PER_W
    sems = (sem0, sem1)
    UR = 8

    def issue(ci, b):
        start = pl.multiple_of(base + ci * CHUNK, 8)
        pltpu.async_copy(data_hbm.at[pl.ds(start, CHUNK)], buf.at[b], sems[b])

    issue(0, 0)
    issue(1, 1)

    def pair_body(pi, accs):
        for b in range(2):
            ci = pi * 2 + b
            pltpu.make_async_copy(
                data_hbm.at[pl.ds(0, CHUNK)], buf.at[b], sems[b]
            ).wait()

            def row_body(r, a):
                for u in range(UR):
                    a = tuple(
                        a[j] + buf[b, r * UR + u, pl.ds(j * L, L)]
                        for j in range(D // L)
                    )
                return a

            accs = lax.fori_loop(0, CHUNK // UR, row_body, accs)

            @pl.when(ci + 2 < NCHUNK)
            def _():
                issue(ci + 2, b)
        return accs

    zero = jnp.zeros((L,), jnp.float32)
    accs = lax.fori_loop(0, NCHUNK // 2, pair_body, (zero,) * (D // L))
    for j in range(D // L):
        accv[pl.ds(j * L, L)] = accs[j]
    pltpu.sync_copy(accv, out_hbm.at[pl.ds(pl.multiple_of(wid * D, 8), D)])


def _tc_body(x_ref, o_ref, acc):
    @pl.when(pl.program_id(0) == 0)
    def _():
        acc[...] = jnp.zeros_like(acc)

    def row_body(r, _):
        acc[...] += x_ref[pl.ds(r * 8, 8), :]
        return 0

    lax.fori_loop(0, TC_BLK // 8, row_body, 0)

    @pl.when(pl.program_id(0) == TC_GRID - 1)
    def _():
        o_ref[...] = jnp.sum(acc[...], axis=0, keepdims=True)


def _tc_partial_sum(data):
    return pl.pallas_call(
        _tc_body,
        grid=(TC_GRID,),
        in_specs=[pl.BlockSpec((TC_BLK, D), lambda i: (TC_START_BLK + i, 0))],
        out_specs=pl.BlockSpec((1, D), lambda i: (0, 0)),
        out_shape=jax.ShapeDtypeStruct((1, D), jnp.float32),
        scratch_shapes=[pltpu.VMEM((8, D), jnp.float32)],
    )(data)


@functools.partial(
    pl.kernel,
    mesh=_mesh,
    out_type=jax.ShapeDtypeStruct((1, D), jnp.float32),
    scratch_types=[
        pltpu.VMEM((NW * D,), jnp.float32),
        pltpu.VMEM((1, D), jnp.float32),
        pltpu.VMEM((1, D), jnp.float32),
    ],
)
def _finalize(part_hbm, tcpart_hbm, out_hbm, buf, tcv, outv):
    wid = lax.axis_index("s") * NC + lax.axis_index("c")

    @pl.when(wid == 0)
    def _():
        pltpu.sync_copy(part_hbm, buf)
        pltpu.sync_copy(tcpart_hbm, tcv)
        inv_n = jnp.float32(1.0 / N)
        for j in range(D // L):
            def row_body(r, a):
                return a + buf[pl.ds(r * D + j * L, L)]

            s = lax.fori_loop(0, NW, row_body, tcv[0, pl.ds(j * L, L)])
            outv[0, pl.ds(j * L, L)] = s * inv_n
        pltpu.sync_copy(outv, out_hbm)


def kernel(data):
    sc_parts = _sc_partial_sums(data)
    tc_part = _tc_partial_sum(data)
    return _finalize(sc_parts, tc_part)


# speedup vs baseline: 2.2597x; 2.2597x over previous
import functools

import jax
import jax.numpy as jnp
from jax import lax
from jax.experimental import pallas as pl
from jax.experimental.pallas import tpu as pltpu
from jax.experimental.pallas import tpu_sc as plsc

N = 320000
D = 128
L = 16
NC = 2
NS = 16
NW = NC * NS

SC_ROWS = 128000
ROWS_PER_W = SC_ROWS // NW
CHUNK = 200
NCHUNK = ROWS_PER_W // CHUNK

TC_ROWS = N - SC_ROWS
TC_BLK = 3200
TC_GRID = TC_ROWS // TC_BLK
TC_START_BLK = SC_ROWS // TC_BLK

_mesh = plsc.VectorSubcoreMesh(core_axis_name="c", subcore_axis_name="s")


@functools.partial(
    pl.kernel,
    mesh=_mesh,
    out_type=jax.ShapeDtypeStruct((NW * D,), jnp.float32),
    scratch_types=[
        pltpu.VMEM((2, CHUNK, D), jnp.float32),
        pltpu.VMEM((D,), jnp.float32),
        pltpu.SemaphoreType.DMA,
        pltpu.SemaphoreType.DMA,
    ],
)
def _sc_partial_sums(data_hbm, out_hbm, buf, accv, sem0, sem1):
    wid = lax.axis_index("s") * NC + lax.axis_index("c")
    base = wid * ROWS_PER_W
    sems = (sem0, sem1)
    UR = 8

    def issue(ci, b):
        start = pl.multiple_of(base + ci * CHUNK, 8)
        pltpu.async_copy(data_hbm.at[pl.ds(start, CHUNK)], buf.at[b], sems[b])

    issue(0, 0)
    issue(1, 1)

    def pair_body(pi, accs):
        for b in range(2):
            ci = pi * 2 + b
            pltpu.make_async_copy(
                data_hbm.at[pl.ds(0, CHUNK)], buf.at[b], sems[b]
            ).wait()

            def row_body(r, a):
                for u in range(UR):
                    a = tuple(
                        a[j] + buf[b, r * UR + u, pl.ds(j * L, L)]
                        for j in range(D // L)
                    )
                return a

            accs = lax.fori_loop(0, CHUNK // UR, row_body, accs)

            @pl.when(ci + 2 < NCHUNK)
            def _():
                issue(ci + 2, b)
        return accs

    zero = jnp.zeros((L,), jnp.float32)
    accs = lax.fori_loop(0, NCHUNK // 2, pair_body, (zero,) * (D // L))
    for j in range(D // L):
        accv[pl.ds(j * L, L)] = accs[j]
    pltpu.sync_copy(accv, out_hbm.at[pl.ds(pl.multiple_of(wid * D, 8), D)])


def _tc_body(x_ref, o_ref, acc):
    @pl.when(pl.program_id(0) == 0)
    def _():
        acc[...] = jnp.zeros_like(acc)

    acc[...] += jnp.sum(
        x_ref[...].reshape(TC_BLK // 8, 8, D), axis=0
    )

    @pl.when(pl.program_id(0) == TC_GRID - 1)
    def _():
        o_ref[...] = jnp.sum(acc[...], axis=0, keepdims=True)


def _tc_partial_sum(data):
    return pl.pallas_call(
        _tc_body,
        grid=(TC_GRID,),
        in_specs=[pl.BlockSpec((TC_BLK, D), lambda i: (TC_START_BLK + i, 0))],
        out_specs=pl.BlockSpec((1, D), lambda i: (0, 0)),
        out_shape=jax.ShapeDtypeStruct((1, D), jnp.float32),
        scratch_shapes=[pltpu.VMEM((8, D), jnp.float32)],
    )(data)


@functools.partial(
    pl.kernel,
    mesh=_mesh,
    out_type=jax.ShapeDtypeStruct((1, D), jnp.float32),
    scratch_types=[
        pltpu.VMEM((NW * D,), jnp.float32),
        pltpu.VMEM((1, D), jnp.float32),
        pltpu.VMEM((1, D), jnp.float32),
    ],
)
def _finalize(part_hbm, tcpart_hbm, out_hbm, buf, tcv, outv):
    wid = lax.axis_index("s") * NC + lax.axis_index("c")

    @pl.when(wid == 0)
    def _():
        pltpu.sync_copy(part_hbm, buf)
        pltpu.sync_copy(tcpart_hbm, tcv)
        inv_n = jnp.float32(1.0 / N)
        for j in range(D // L):
            def row_body(r, a):
                return a + buf[pl.ds(r * D + j * L, L)]

            s = lax.fori_loop(0, NW, row_body, tcv[0, pl.ds(j * L, L)])
            outv[0, pl.ds(j * L, L)] = s * inv_n
        pltpu.sync_copy(outv, out_hbm)


def kernel(data):
    sc_parts = _sc_partial_sums(data)
    tc_part = _tc_partial_sum(data)
    return _finalize(sc_parts, tc_part)


# speedup vs baseline: 2.5665x; 1.1358x over previous
import functools

import jax
import jax.numpy as jnp
from jax import lax
from jax.experimental import pallas as pl
from jax.experimental.pallas import tpu as pltpu
from jax.experimental.pallas import tpu_sc as plsc

N = 320000
D = 128
L = 16
NC = 2
NS = 16
NW = NC * NS

SC_ROWS = 166400
ROWS_PER_W = SC_ROWS // NW
CHUNK = 200
NCHUNK = ROWS_PER_W // CHUNK

TC_ROWS = N - SC_ROWS
TC_BLK = 3200
TC_GRID = TC_ROWS // TC_BLK
TC_START_BLK = SC_ROWS // TC_BLK

_mesh = plsc.VectorSubcoreMesh(core_axis_name="c", subcore_axis_name="s")


@functools.partial(
    pl.kernel,
    mesh=_mesh,
    out_type=jax.ShapeDtypeStruct((NW * D,), jnp.float32),
    scratch_types=[
        pltpu.VMEM((2, CHUNK, D), jnp.float32),
        pltpu.VMEM((D,), jnp.float32),
        pltpu.SemaphoreType.DMA,
        pltpu.SemaphoreType.DMA,
    ],
)
def _sc_partial_sums(data_hbm, out_hbm, buf, accv, sem0, sem1):
    wid = lax.axis_index("s") * NC + lax.axis_index("c")
    base = wid * ROWS_PER_W
    sems = (sem0, sem1)
    UR = 8

    def issue(ci, b):
        start = pl.multiple_of(base + ci * CHUNK, 8)
        pltpu.async_copy(data_hbm.at[pl.ds(start, CHUNK)], buf.at[b], sems[b])

    issue(0, 0)
    issue(1, 1)

    def pair_body(pi, accs):
        for b in range(2):
            ci = pi * 2 + b
            pltpu.make_async_copy(
                data_hbm.at[pl.ds(0, CHUNK)], buf.at[b], sems[b]
            ).wait()

            def row_body(r, a):
                for u in range(UR):
                    a = tuple(
                        a[j] + buf[b, r * UR + u, pl.ds(j * L, L)]
                        for j in range(D // L)
                    )
                return a

            accs = lax.fori_loop(0, CHUNK // UR, row_body, accs)

            @pl.when(ci + 2 < NCHUNK)
            def _():
                issue(ci + 2, b)
        return accs

    zero = jnp.zeros((L,), jnp.float32)
    accs = lax.fori_loop(0, NCHUNK // 2, pair_body, (zero,) * (D // L))
    for j in range(D // L):
        accv[pl.ds(j * L, L)] = accs[j]
    pltpu.sync_copy(accv, out_hbm.at[pl.ds(pl.multiple_of(wid * D, 8), D)])


def _tc_body(x_ref, o_ref, acc):
    @pl.when(pl.program_id(0) == 0)
    def _():
        acc[...] = jnp.zeros_like(acc)

    acc[...] += jnp.sum(
        x_ref[...].reshape(TC_BLK // 8, 8, D), axis=0
    )

    @pl.when(pl.program_id(0) == TC_GRID - 1)
    def _():
        o_ref[...] = jnp.sum(acc[...], axis=0, keepdims=True)


def _tc_partial_sum(data):
    return pl.pallas_call(
        _tc_body,
        grid=(TC_GRID,),
        in_specs=[pl.BlockSpec((TC_BLK, D), lambda i: (TC_START_BLK + i, 0))],
        out_specs=pl.BlockSpec((1, D), lambda i: (0, 0)),
        out_shape=jax.ShapeDtypeStruct((1, D), jnp.float32),
        scratch_shapes=[pltpu.VMEM((8, D), jnp.float32)],
    )(data)


def _combine_body(parts_ref, tcpart_ref, o_ref):
    total = jnp.sum(parts_ref[...], axis=0, keepdims=True) + tcpart_ref[...]
    o_ref[...] = total * jnp.float32(1.0 / N)


def _combine(sc_parts, tc_part):
    return pl.pallas_call(
        _combine_body,
        out_shape=jax.ShapeDtypeStruct((1, D), jnp.float32),
    )(sc_parts.reshape(NW, D), tc_part)


def kernel(data):
    sc_parts = _sc_partial_sums(data)
    tc_part = _tc_partial_sum(data)
    return _combine(sc_parts, tc_part)
